# Initial kernel scaffold; baseline (speedup 1.0000x reference)
import jax
import jax.numpy as jnp
from jax.experimental import pallas as pl


def kernel(cls_ctx_per_id, cls_vector, compound_prompts_text, compound_per_id_prompts_text, token_prefix, token_suffix, label):
    raise NotImplementedError("write your pallas kernel here")


# speedup vs baseline: 1.6445x; 1.6445x over previous
import functools

import jax
import jax.numpy as jnp
from jax import lax
from jax.experimental import pallas as pl
from jax.experimental.pallas import tpu as pltpu
from jax.experimental.pallas import tpu_sc as plsc

_NUM_CLASS = 100000
_D = 512
_B = 1024
_DEPTH_M1 = 8
_SEQ = 77
_NC = 2
_NS = 16
_NW = _NC * _NS
_BPW = _B // _NW


def _sc_gather_body(cls_hbm, deep_hbm, idx_hbm, *rest):
    outs = rest[:9]
    idx_v, rows_v, sem = rest[9:]
    wid = lax.axis_index("s") * _NC + lax.axis_index("c")
    base = wid * _BPW
    for t in range(9):
        pltpu.sync_copy(idx_hbm.at[pl.ds(t * _B + base, _BPW)], idx_v)
        src = cls_hbm if t == 0 else deep_hbm
        pltpu.async_copy(src.at[idx_v], rows_v, sem).wait()
        pltpu.sync_copy(rows_v, outs[t].at[pl.ds(base, _BPW)])


_sc_gather = functools.partial(
    pl.kernel,
    out_type=[jax.ShapeDtypeStruct((_B, _D), jnp.float32) for _ in range(9)],
    mesh=plsc.VectorSubcoreMesh(core_axis_name="c", subcore_axis_name="s"),
    scratch_types=[
        pltpu.VMEM((_BPW,), jnp.int32),
        pltpu.VMEM((_BPW, _D), jnp.float32),
        pltpu.SemaphoreType.DMA,
    ],
)(_sc_gather_body)


_BB = 64
_PF = 5 * _D
_CV = 2 * _D
_SF = 69 * _D
_W = _SEQ * _D


def _tc_assemble_body(g_ref, pf_ref, cv_ref, sf_ref, o_ref):
    o_ref[:, 0:_PF] = jnp.broadcast_to(pf_ref[...], (_BB, _PF))
    o_ref[:, _PF:_PF + _CV] = jnp.broadcast_to(cv_ref[...], (_BB, _CV))
    o_ref[:, _PF + _CV:_PF + _CV + _D] = g_ref[...]
    o_ref[:, _PF + _CV + _D:_W] = jnp.broadcast_to(sf_ref[...], (_BB, _SF))


def _tc_assemble(g, pf, cv, sf):
    return pl.pallas_call(
        _tc_assemble_body,
        grid=(_B // _BB,),
        in_specs=[
            pl.BlockSpec((_BB, _D), lambda i: (i, 0)),
            pl.BlockSpec((1, _PF), lambda i: (0, 0)),
            pl.BlockSpec((1, _CV), lambda i: (0, 0)),
            pl.BlockSpec((1, _SF), lambda i: (0, 0)),
        ],
        out_specs=pl.BlockSpec((_BB, _W), lambda i: (i, 0)),
        out_shape=jax.ShapeDtypeStruct((_B, _W), jnp.float32),
    )(g, pf, cv, sf)


def kernel(cls_ctx_per_id, cls_vector, compound_prompts_text,
           compound_per_id_prompts_text, token_prefix, token_suffix, label):
    cls2d = cls_ctx_per_id.reshape(_NUM_CLASS, _D)
    deep2d = compound_per_id_prompts_text.reshape(_DEPTH_M1 * _NUM_CLASS, _D)
    offs = jnp.array([0] + [j * _NUM_CLASS for j in range(_DEPTH_M1)],
                     dtype=jnp.int32)
    idx_all = (offs[:, None] + label.astype(jnp.int32)[None, :]).reshape(-1)

    gathered = _sc_gather(cls2d, deep2d, idx_all)

    prompts2d = _tc_assemble(
        gathered[0],
        token_prefix.reshape(1, _PF),
        cls_vector.reshape(1, _CV),
        token_suffix.reshape(1, _SF),
    )
    prompts = prompts2d.reshape(_B, _SEQ, _D)

    compound_prompts = tuple(compound_prompts_text[i] for i in range(_DEPTH_M1))
    deep_per_id = tuple(gathered[1 + i].reshape(_B, 1, _D)
                        for i in range(_DEPTH_M1))
    return (prompts, compound_prompts, deep_per_id)


# speedup vs baseline: 13.2365x; 8.0491x over previous
import functools

import jax
import jax.numpy as jnp
from jax import lax
from jax.experimental import pallas as pl
from jax.experimental.pallas import tpu as pltpu
from jax.experimental.pallas import tpu_sc as plsc

_NUM_CLASS = 100000
_D = 512
_B = 1024
_DEPTH_M1 = 8
_SEQ = 77
_NC = 2
_NS = 16
_NW = _NC * _NS
_BPW = _B // _NW


def _sc_gather_body(cls_hbm, deep_hbm, idx_hbm, *rest):
    outs = rest[:9]
    idx_v, rows_v, sem = rest[9:]
    wid = lax.axis_index("s") * _NC + lax.axis_index("c")
    base = wid * _BPW
    for t in range(9):
        pltpu.sync_copy(idx_hbm.at[pl.ds(t * _B + base, _BPW)], idx_v)
        src = cls_hbm if t == 0 else deep_hbm
        pltpu.async_copy(src.at[idx_v], rows_v, sem).wait()
        pltpu.sync_copy(rows_v, outs[t].at[pl.ds(base, _BPW)])


_sc_gather = functools.partial(
    pl.kernel,
    out_type=[jax.ShapeDtypeStruct((_B, 1, _D), jnp.float32) for _ in range(9)],
    mesh=plsc.VectorSubcoreMesh(core_axis_name="c", subcore_axis_name="s"),
    scratch_types=[
        pltpu.VMEM((_BPW,), jnp.int32),
        pltpu.VMEM((_BPW, 1, _D), jnp.float32),
        pltpu.SemaphoreType.DMA,
    ],
)(_sc_gather_body)


_BB = 64


def _tc_assemble_body(g_ref, pf_ref, cv_ref, sf_ref, o_ref):
    pf = pf_ref[0]
    cv = cv_ref[...]
    sf = sf_ref[0]
    o_ref[:, 0:5, :] = jnp.broadcast_to(pf[None], (_BB, 5, _D))
    o_ref[:, 5:7, :] = jnp.broadcast_to(cv[None], (_BB, 2, _D))
    o_ref[:, 7:8, :] = g_ref[...]
    o_ref[:, 8:_SEQ, :] = jnp.broadcast_to(sf[None], (_BB, _SEQ - 8, _D))


def _tc_assemble(g, pf, cv, sf):
    return pl.pallas_call(
        _tc_assemble_body,
        grid=(_B // _BB,),
        in_specs=[
            pl.BlockSpec((_BB, 1, _D), lambda i: (i, 0, 0)),
            pl.BlockSpec((1, 5, _D), lambda i: (0, 0, 0)),
            pl.BlockSpec((2, _D), lambda i: (0, 0)),
            pl.BlockSpec((1, _SEQ - 8, _D), lambda i: (0, 0, 0)),
        ],
        out_specs=pl.BlockSpec((_BB, _SEQ, _D), lambda i: (i, 0, 0)),
        out_shape=jax.ShapeDtypeStruct((_B, _SEQ, _D), jnp.float32),
    )(g, pf, cv, sf)


def kernel(cls_ctx_per_id, cls_vector, compound_prompts_text,
           compound_per_id_prompts_text, token_prefix, token_suffix, label):
    deep_flat = compound_per_id_prompts_text.reshape(
        _DEPTH_M1 * _NUM_CLASS, 1, _D)
    lbl = label.astype(jnp.int32)
    offs = jnp.array([0] + [j * _NUM_CLASS for j in range(_DEPTH_M1)],
                     dtype=jnp.int32)
    idx_all = (offs[:, None] + lbl[None, :]).reshape(-1)

    gathered = _sc_gather(cls_ctx_per_id, deep_flat, idx_all)

    prompts = _tc_assemble(gathered[0], token_prefix, cls_vector, token_suffix)

    compound_prompts = tuple(compound_prompts_text[i] for i in range(_DEPTH_M1))
    deep_per_id = tuple(gathered[1 + i] for i in range(_DEPTH_M1))
    return (prompts, compound_prompts, deep_per_id)


# speedup vs baseline: 27.0825x; 2.0461x over previous
import functools

import jax
import jax.numpy as jnp
from jax import lax
from jax.experimental import pallas as pl
from jax.experimental.pallas import tpu as pltpu
from jax.experimental.pallas import tpu_sc as plsc

_NUM_CLASS = 100000
_D = 512
_B = 1024
_DEPTH_M1 = 8
_SEQ = 77
_NC = 2
_NS = 16
_NW = _NC * _NS
_BPW = _B // _NW

_SC_MESH = plsc.VectorSubcoreMesh(core_axis_name="c", subcore_axis_name="s")


def _worker_base():
    wid = lax.axis_index("s") * _NC + lax.axis_index("c")
    return wid * _BPW


def _sc_gather_cls_body(tab_hbm, idx_hbm, out, idx_v, rows_v, sem):
    base = _worker_base()
    pltpu.sync_copy(idx_hbm.at[pl.ds(base, _BPW)], idx_v)
    pltpu.async_copy(tab_hbm.at[idx_v], rows_v, sem).wait()
    pltpu.sync_copy(rows_v, out.at[pl.ds(base, _BPW)])


_sc_gather_cls = functools.partial(
    pl.kernel,
    out_type=jax.ShapeDtypeStruct((_B, 1, _D), jnp.float32),
    mesh=_SC_MESH,
    scratch_types=[
        pltpu.VMEM((_BPW,), jnp.int32),
        pltpu.VMEM((_BPW, 1, _D), jnp.float32),
        pltpu.SemaphoreType.DMA,
    ],
)(_sc_gather_cls_body)


def _sc_gather_deep_body(tab_hbm, idx_hbm, *rest):
    outs = rest[:_DEPTH_M1]
    idx_v, rows_v, sem = rest[_DEPTH_M1:]
    base = _worker_base()
    for t in range(_DEPTH_M1):
        pltpu.sync_copy(idx_hbm.at[pl.ds(t * _B + base, _BPW)], idx_v)
        pltpu.async_copy(tab_hbm.at[idx_v], rows_v, sem).wait()
        pltpu.sync_copy(rows_v, outs[t].at[pl.ds(base, _BPW)])


_sc_gather_deep = functools.partial(
    pl.kernel,
    out_type=[jax.ShapeDtypeStruct((_B, 1, _D), jnp.float32)
              for _ in range(_DEPTH_M1)],
    mesh=_SC_MESH,
    scratch_types=[
        pltpu.VMEM((_BPW,), jnp.int32),
        pltpu.VMEM((_BPW, 1, _D), jnp.float32),
        pltpu.SemaphoreType.DMA,
    ],
)(_sc_gather_deep_body)


def _tc_assemble_body(r_ref, g_ref, o_ref):
    s = pl.program_id(0)

    @pl.when(s == 7)
    def _():
        o_ref[0] = g_ref[:, 0, :]

    @pl.when(s != 7)
    def _():
        o_ref[0] = jnp.broadcast_to(r_ref[0, 0], (_B, _D))


def _tc_assemble(rows, g):
    return pl.pallas_call(
        _tc_assemble_body,
        grid=(_SEQ,),
        in_specs=[
            pl.BlockSpec((1, 1, _D), lambda s: (s, 0, 0)),
            pl.BlockSpec((_B, 1, _D), lambda s: (0, 0, 0)),
        ],
        out_specs=pl.BlockSpec((1, _B, _D), lambda s: (s, 0, 0)),
        out_shape=jax.ShapeDtypeStruct((_SEQ, _B, _D), jnp.float32),
    )(rows, g)


def kernel(cls_ctx_per_id, cls_vector, compound_prompts_text,
           compound_per_id_prompts_text, token_prefix, token_suffix, label):
    deep_flat = compound_per_id_prompts_text.reshape(
        _DEPTH_M1 * _NUM_CLASS, 1, _D)
    lbl = label.astype(jnp.int32)
    offs = jnp.arange(_DEPTH_M1, dtype=jnp.int32) * _NUM_CLASS
    idx_deep = (offs[:, None] + lbl[None, :]).reshape(-1)

    g_cls = _sc_gather_cls(cls_ctx_per_id, lbl)
    g_deep = _sc_gather_deep(deep_flat, idx_deep)

    rows = jnp.concatenate(
        [token_prefix[0], cls_vector, cls_vector[:1], token_suffix[0]],
        axis=0).reshape(_SEQ, 1, _D)

    prompts = _tc_assemble(rows, g_cls).transpose(1, 0, 2)

    compound_prompts = tuple(compound_prompts_text[i] for i in range(_DEPTH_M1))
    return (prompts, compound_prompts, g_deep)
